# baseline (speedup 1.0000x reference)
import functools

import jax
import jax.numpy as jnp
from jax import lax
from jax.experimental import pallas as pl
from jax.experimental.pallas import tpu as pltpu
from jax.experimental.pallas import tpu_sc as plsc

NC = 2
NS = 16
NW = NC * NS
NBLK = 32
NBS = 3
L = 16


def kernel(phone, table):
    B, S = phone.shape
    V, D = table.shape
    per_w = B // NW
    n_blk = per_w // NBLK
    n_full = S // L
    tail = S - n_full * L
    idx3 = phone.reshape(NW, per_w, S)
    table_p = jnp.pad(table, ((0, 0), (0, 128 - D)))

    mesh = plsc.VectorSubcoreMesh(core_axis_name="c", subcore_axis_name="s")

    @functools.partial(
        pl.kernel,
        mesh=mesh,
        out_type=jax.ShapeDtypeStruct((B, S, D), jnp.float32),
        compiler_params=pltpu.CompilerParams(needs_layout_passes=False),
        scratch_types=[
            pltpu.VMEM((V, 128), jnp.float32),
            pltpu.VMEM((2, NBLK, S), jnp.int32),
            pltpu.VMEM((NBS, S, D), jnp.float32),
            pltpu.SemaphoreType.DMA((2,)),
            pltpu.SemaphoreType.DMA((NBS,)),
        ],
    )
    def emb(idx_hbm, table_hbm, out_hbm, tbl_v, ibuf, cbuf, isem, ssem):
        wid = lax.axis_index("s") * NC + lax.axis_index("c")
        base = wid * per_w

        def idx_block(m):
            bm = m % 2
            return (
                idx_hbm.at[wid, pl.ds(m * NBLK, NBLK)],
                ibuf.at[bm],
                isem.at[bm],
            )

        pltpu.async_copy(*idx_block(0))
        pltpu.sync_copy(table_hbm, tbl_v)
        iota = lax.iota(jnp.int32, L)

        for m in range(n_blk):
            if m + 1 < n_blk:
                pltpu.async_copy(*idx_block(m + 1))
            pltpu.make_async_copy(*idx_block(m)).wait()
            bm = m % 2

            def body(jj, carry):
                j = m * NBLK + jj
                bs = lax.rem(j, NBS)

                @pl.when(j >= NBS)
                def _():
                    pltpu.make_async_copy(
                        cbuf.at[bs], out_hbm.at[base + j - NBS], ssem.at[bs]
                    ).wait()

                slab = cbuf.at[bs]

                def group(start):
                    v_idx = ibuf[bm, jj, pl.ds(start, L)]
                    rows = start + iota
                    for d in range(D):
                        col = jnp.full((L,), d, jnp.int32)
                        val = plsc.load_gather(tbl_v, [v_idx, col])
                        plsc.store_scatter(slab, [rows, col], val)

                def fullg(g, c):
                    group(g * L)
                    return c

                lax.fori_loop(0, n_full, fullg, 0)
                if tail:
                    group(S - L)

                pltpu.async_copy(slab, out_hbm.at[base + j], ssem.at[bs])
                return carry

            lax.fori_loop(0, NBLK, body, 0)

        for i in range(NBS):
            j = per_w - NBS + i
            pltpu.make_async_copy(
                cbuf.at[j % NBS], out_hbm.at[base + j], ssem.at[j % NBS]
            ).wait()

    return emb(idx3, table_p)


# speedup vs baseline: 3.2089x; 3.2089x over previous
import functools

import jax
import jax.numpy as jnp
from jax import lax
from jax.experimental import pallas as pl
from jax.experimental.pallas import tpu as pltpu
from jax.experimental.pallas import tpu_sc as plsc

NC = 2
NS = 16
NW = NC * NS
NBLK = 32
NBS = 3
L = 16


def kernel(phone, table):
    B, S = phone.shape
    V, D = table.shape
    per_w = B // NW
    n_blk = per_w // NBLK
    n_full = S // L
    tail = S - n_full * L
    idx3 = phone.reshape(NW, per_w, S)
    table_p = jnp.pad(table, ((0, 0), (0, 128 - D)))

    mesh = plsc.VectorSubcoreMesh(core_axis_name="c", subcore_axis_name="s")

    @functools.partial(
        pl.kernel,
        mesh=mesh,
        out_type=jax.ShapeDtypeStruct((B, S, D), jnp.float32),
        compiler_params=pltpu.CompilerParams(needs_layout_passes=False),
        scratch_types=[
            pltpu.VMEM((V, 128), jnp.float32),
            pltpu.VMEM((2, NBLK, S), jnp.int32),
            pltpu.VMEM((NBS, S, D), jnp.float32),
            pltpu.SemaphoreType.DMA((2,)),
            pltpu.SemaphoreType.DMA((NBS,)),
        ],
    )
    def emb(idx_hbm, table_hbm, out_hbm, tbl_v, ibuf, cbuf, isem, ssem):
        wid = lax.axis_index("s") * NC + lax.axis_index("c")
        base = wid * per_w

        def idx_block(m):
            bm = m % 2
            return (
                idx_hbm.at[wid, pl.ds(m * NBLK, NBLK)],
                ibuf.at[bm],
                isem.at[bm],
            )

        pltpu.async_copy(*idx_block(0))
        pltpu.sync_copy(table_hbm, tbl_v)
        iota = lax.iota(jnp.int32, L)

        for m in range(n_blk):
            if m + 1 < n_blk:
                pltpu.async_copy(*idx_block(m + 1))
            pltpu.make_async_copy(*idx_block(m)).wait()
            bm = m % 2

            def body(jj, carry):
                j = m * NBLK + jj
                bs = lax.rem(j, NBS)

                @pl.when(j >= NBS)
                def _():
                    pltpu.make_async_copy(
                        cbuf.at[bs], out_hbm.at[base + j - NBS], ssem.at[bs]
                    ).wait()

                slab = cbuf.at[bs]

                def group(start):
                    v_idx = ibuf[bm, jj, pl.ds(start, L)]
                    rows = start + iota
                    for d0 in range(D):
                        col = d0 + iota
                        col = jnp.where(col < D, col, col - D)
                        val = plsc.load_gather(tbl_v, [v_idx, col])
                        plsc.store_scatter(slab, [rows, col], val)

                def fullg(g, c):
                    group(g * L)
                    return c

                lax.fori_loop(0, n_full, fullg, 0)
                if tail:
                    group(S - L)

                pltpu.async_copy(slab, out_hbm.at[base + j], ssem.at[bs])
                return carry

            lax.fori_loop(0, NBLK, body, 0)

        for i in range(NBS):
            j = per_w - NBS + i
            pltpu.make_async_copy(
                cbuf.at[j % NBS], out_hbm.at[base + j], ssem.at[j % NBS]
            ).wait()

    return emb(idx3, table_p)
